# baseline (speedup 1.0000x reference)
import functools

import jax
import jax.numpy as jnp
from jax import lax
from jax.experimental import pallas as pl
from jax.experimental.pallas import tpu as pltpu
from jax.experimental.pallas import tpu_sc as plsc

N = 10000
E = 320000
D_IN = 128
D_HID = 64
N_CLASSES = 4

N2 = 10240
DP = 128

NC = 2
NS = 16
NW = NC * NS
CH = 128
NCH = 80
EPT = NCH * CH
E2 = NW * EPT
RPT = N2 // NS
RCH = 32
RD = 320

_MESH = plsc.VectorSubcoreMesh(core_axis_name="c", subcore_axis_name="s")


@functools.partial(
    pl.kernel,
    out_type=jax.ShapeDtypeStruct((NC, N2, 16), jnp.float32),
    mesh=_MESH,
    scratch_types=[
        pltpu.VMEM_SHARED((N2, 16), jnp.float32),
        pltpu.VMEM((2, CH), jnp.int32),
        pltpu.VMEM((CH, 16), jnp.float32),
        pltpu.VMEM((RD, 16), jnp.float32),
        pltpu.SemaphoreType.DMA,
        pltpu.SemaphoreType.DMA,
    ],
)
def _deg_kernel(dst_hbm, out_hbm, accum, dstv, ones_rows, zbuf, sd0, sd1):
    c = lax.axis_index("c")
    s = lax.axis_index("s")
    wid = c * NS + s
    sd = (sd0, sd1)

    one = jnp.full((16,), 1.0, dtype=jnp.float32)
    zero = jnp.full((16,), 0.0, dtype=jnp.float32)

    @pl.loop(0, CH)
    def _(i):
        ones_rows[i, :] = one

    @pl.loop(0, RD)
    def _(i):
        zbuf[i, :] = zero

    r0 = s * RPT
    for k in range(RPT // RD):
        pltpu.sync_copy(zbuf, accum.at[pl.ds(r0 + k * RD, RD)])
    plsc.subcore_barrier()

    base0 = wid * EPT
    for j in range(2):
        pltpu.async_copy(dst_hbm.at[pl.ds(base0 + j * CH, CH)],
                         dstv.at[j], sd[j])

    @pl.loop(0, NCH, step=2)
    def _(g):
        for j in range(2):
            ci = g + j
            pltpu.make_async_copy(dst_hbm.at[pl.ds(0, CH)],
                                  dstv.at[j], sd[j]).wait()
            pltpu.sync_copy(ones_rows, accum.at[dstv.at[j]], add=True)

            @pl.when(ci + 2 < NCH)
            def _():
                pltpu.async_copy(
                    dst_hbm.at[pl.ds(base0 + (ci + 2) * CH, CH)],
                    dstv.at[j], sd[j])

    plsc.subcore_barrier()
    for k in range(RPT // RD):
        pltpu.sync_copy(accum.at[pl.ds(r0 + k * RD, RD)], zbuf)
        pltpu.sync_copy(zbuf, out_hbm.at[c, pl.ds(r0 + k * RD, RD)])


@functools.partial(
    pl.kernel,
    out_type=jax.ShapeDtypeStruct((NC, N2, DP), jnp.float32),
    mesh=_MESH,
    scratch_types=[
        pltpu.VMEM_SHARED((N2, DP), jnp.float32),
        pltpu.VMEM((4, CH), jnp.int32),
        pltpu.VMEM((4, CH), jnp.int32),
        pltpu.VMEM((4, 2, CH // 2), jnp.int32),
        pltpu.VMEM((2, CH, DP), jnp.float32),
        pltpu.VMEM((2, RCH, DP), jnp.float32),
        pltpu.SemaphoreType.DMA, pltpu.SemaphoreType.DMA,
        pltpu.SemaphoreType.DMA, pltpu.SemaphoreType.DMA,
        pltpu.SemaphoreType.DMA, pltpu.SemaphoreType.DMA,
        pltpu.SemaphoreType.DMA, pltpu.SemaphoreType.DMA,
        pltpu.SemaphoreType.DMA, pltpu.SemaphoreType.DMA,
        pltpu.SemaphoreType.DMA, pltpu.SemaphoreType.DMA,
        pltpu.SemaphoreType.DMA, pltpu.SemaphoreType.DMA,
        pltpu.SemaphoreType.DMA, pltpu.SemaphoreType.DMA,
    ],
)
def _agg_kernel(h_hbm, src_hbm, dst_hbm, out_hbm,
                accum, srcv, dstv, dsth, rows, zbuf,
                ss0, ss1, ss2, ss3, sd0, sd1, sd2, sd3,
                sg0, sg1, sg2, sg3, sc0, sc1, sw0, sw1):
    c = lax.axis_index("c")
    s = lax.axis_index("s")
    wid = c * NS + s
    ssrc = (ss0, ss1, ss2, ss3)
    sdst = (sd0, sd1, sd2, sd3)
    sgat = ((sg0, sg1), (sg2, sg3))
    ssca = (sc0, sc1)
    swr = (sw0, sw1)
    CH2 = CH // 2

    zero = jnp.full((16,), 0.0, dtype=jnp.float32)

    @pl.loop(0, RCH)
    def _(i):
        @pl.loop(0, DP, step=16)
        def _(j):
            zbuf[0, i, pl.ds(j, 16)] = zero

    r0 = s * RPT
    for k in range(RPT // RCH):
        pltpu.sync_copy(zbuf.at[0], accum.at[pl.ds(r0 + k * RCH, RCH)])
    plsc.subcore_barrier()

    base0 = wid * EPT

    def idx_start(chunk, j):
        pltpu.async_copy(src_hbm.at[pl.ds(base0 + chunk * CH, CH)],
                         srcv.at[j], ssrc[j])
        pltpu.async_copy(dst_hbm.at[pl.ds(base0 + chunk * CH, CH)],
                         dstv.at[j], sdst[j])

    def src_wait(j):
        pltpu.make_async_copy(src_hbm.at[pl.ds(0, CH)],
                              srcv.at[j], ssrc[j]).wait()

    def dst_wait(j):
        pltpu.make_async_copy(dst_hbm.at[pl.ds(0, CH)],
                              dstv.at[j], sdst[j]).wait()

    def gather_start(j, r):
        for hh in range(2):
            pltpu.async_copy(h_hbm.at[srcv.at[j, pl.ds(hh * CH2, CH2)]],
                             rows.at[r, pl.ds(hh * CH2, CH2)], sgat[r][hh])

    def gather_wait(r):
        for hh in range(2):
            pltpu.make_async_copy(h_hbm.at[pl.ds(0, CH2)],
                                  rows.at[r, pl.ds(hh * CH2, CH2)],
                                  sgat[r][hh]).wait()

    def scatter(j, r):
        @pl.loop(0, CH2, step=16)
        def _(q):
            dsth[j, 0, pl.ds(q, 16)] = dstv[j, pl.ds(q, 16)]
            dsth[j, 1, pl.ds(q, 16)] = dstv[j, pl.ds(CH2 + q, 16)]

        for hh in range(2):
            pltpu.async_copy(rows.at[r, pl.ds(hh * CH2, CH2)],
                             accum.at[dsth.at[j, hh]], ssca[hh], add=True)
        for hh in range(2):
            pltpu.make_async_copy(rows.at[r, pl.ds(hh * CH2, CH2)],
                                  accum.at[pl.ds(0, CH2)], ssca[hh]).wait()

    for j in range(4):
        idx_start(j, j)
    src_wait(0)
    gather_start(0, 0)

    @pl.loop(0, NCH, step=4)
    def _(g):
        for j in range(4):
            ci = g + j
            jn = (j + 1) % 4
            r = j % 2
            rn = (j + 1) % 2

            @pl.when(ci + 1 < NCH)
            def _():
                src_wait(jn)
                gather_start(jn, rn)

            gather_wait(r)
            dst_wait(j)
            scatter(j, r)

            @pl.when(ci + 4 < NCH)
            def _():
                idx_start(ci + 4, j)

    plsc.subcore_barrier()
    for k in range(RPT // RCH):
        b = k % 2
        if k >= 2:
            pltpu.make_async_copy(zbuf.at[b],
                                  out_hbm.at[c, pl.ds(r0, RCH)],
                                  swr[b]).wait()
        pltpu.sync_copy(accum.at[pl.ds(r0 + k * RCH, RCH)], zbuf.at[b])
        pltpu.async_copy(zbuf.at[b],
                         out_hbm.at[c, pl.ds(r0 + k * RCH, RCH)], swr[b])
    for b in range(2):
        pltpu.make_async_copy(zbuf.at[b], out_hbm.at[c, pl.ds(r0, RCH)],
                              swr[b]).wait()


_RB = 640
_NB = N2 // _RB


def _pre_body(degp_ref, x_ref, w0_ref, dis_ref, hp_ref):
    deg = 1.0 + degp_ref[0, :, 0:1] + degp_ref[1, :, 0:1]
    dis = lax.rsqrt(deg)
    dis_ref[...] = dis
    h = jnp.dot(x_ref[...], w0_ref[...], preferred_element_type=jnp.float32)
    hp_ref[...] = h * dis


def _pre_kernel(degp, x, w0):
    return pl.pallas_call(
        _pre_body,
        grid=(_NB,),
        in_specs=[
            pl.BlockSpec((NC, _RB, 16), lambda i: (0, i, 0)),
            pl.BlockSpec((_RB, D_IN), lambda i: (i, 0)),
            pl.BlockSpec((D_IN, DP), lambda i: (0, 0)),
        ],
        out_specs=[
            pl.BlockSpec((_RB, 1), lambda i: (i, 0)),
            pl.BlockSpec((_RB, DP), lambda i: (i, 0)),
        ],
        out_shape=[
            jax.ShapeDtypeStruct((N2, 1), jnp.float32),
            jax.ShapeDtypeStruct((N2, DP), jnp.float32),
        ],
    )(degp, x, w0)


def _mid_body(p_ref, hp_ref, dis_ref, b_ref, w_ref, out_ref):
    dis = dis_ref[...]
    act = (p_ref[0] + p_ref[1] + hp_ref[...]) * dis + b_ref[...]
    act = jnp.maximum(act, 0.0)
    h = jnp.dot(act, w_ref[...], preferred_element_type=jnp.float32)
    out_ref[...] = h * dis


def _mid_kernel(p, hp, dis, b, w):
    return pl.pallas_call(
        _mid_body,
        grid=(_NB,),
        in_specs=[
            pl.BlockSpec((NC, _RB, DP), lambda i: (0, i, 0)),
            pl.BlockSpec((_RB, DP), lambda i: (i, 0)),
            pl.BlockSpec((_RB, 1), lambda i: (i, 0)),
            pl.BlockSpec((1, DP), lambda i: (0, 0)),
            pl.BlockSpec((DP, DP), lambda i: (0, 0)),
        ],
        out_specs=pl.BlockSpec((_RB, DP), lambda i: (i, 0)),
        out_shape=jax.ShapeDtypeStruct((N2, DP), jnp.float32),
    )(p, hp, dis, b, w)


def _final_body(p_ref, hp_ref, dis_ref, b_ref, wl_ref, bl_ref, out_ref):
    dis = dis_ref[...]
    act = (p_ref[0] + p_ref[1] + hp_ref[...]) * dis + b_ref[...]
    act = jnp.maximum(act, 0.0)
    logits = jnp.dot(act, wl_ref[...], preferred_element_type=jnp.float32)
    logits = logits + bl_ref[...]
    m = jnp.max(logits, axis=-1, keepdims=True)
    zs = logits - m
    lse = jnp.log(jnp.sum(jnp.exp(zs), axis=-1, keepdims=True))
    out_ref[...] = zs - lse


def _final_kernel(p, hp, dis, b, wl, bl):
    return pl.pallas_call(
        _final_body,
        grid=(_NB,),
        in_specs=[
            pl.BlockSpec((NC, _RB, DP), lambda i: (0, i, 0)),
            pl.BlockSpec((_RB, DP), lambda i: (i, 0)),
            pl.BlockSpec((_RB, 1), lambda i: (i, 0)),
            pl.BlockSpec((1, DP), lambda i: (0, 0)),
            pl.BlockSpec((DP, N_CLASSES), lambda i: (0, 0)),
            pl.BlockSpec((1, N_CLASSES), lambda i: (0, 0)),
        ],
        out_specs=pl.BlockSpec((_RB, N_CLASSES), lambda i: (i, 0)),
        out_shape=jax.ShapeDtypeStruct((N2, N_CLASSES), jnp.float32),
    )(p, hp, dis, b, wl, bl)


def _pad_w(w):
    return jnp.pad(w, ((0, DP - w.shape[0]), (0, DP - w.shape[1])))


def kernel(x, edge_index, W0, b0, W1, b1, W2, b2, W3, b3, Wl, bl):
    pad_ids = (jnp.arange(E2 - E, dtype=jnp.int32) % (N2 - N)) + N
    src = jnp.concatenate([edge_index[0], pad_ids])
    dst = jnp.concatenate([edge_index[1], pad_ids])
    xp = jnp.pad(x, ((0, N2 - N), (0, 0)))
    w0p = jnp.pad(W0, ((0, 0), (0, DP - D_HID)))
    w1p, w2p, w3p = _pad_w(W1), _pad_w(W2), _pad_w(W3)
    wlp = jnp.pad(Wl, ((0, DP - D_HID), (0, 0)))
    b0r = jnp.pad(b0, (0, DP - D_HID)).reshape(1, DP)
    b1r = jnp.pad(b1, (0, DP - D_HID)).reshape(1, DP)
    b2r = jnp.pad(b2, (0, DP - D_HID)).reshape(1, DP)
    b3r = jnp.pad(b3, (0, DP - D_HID)).reshape(1, DP)
    blr = bl.reshape(1, N_CLASSES)

    degp = _deg_kernel(dst)
    dis, hp0 = _pre_kernel(degp, xp, w0p)
    p = _agg_kernel(hp0, src, dst)
    hp1 = _mid_kernel(p, hp0, dis, b0r, w1p)
    p = _agg_kernel(hp1, src, dst)
    hp2 = _mid_kernel(p, hp1, dis, b1r, w2p)
    p = _agg_kernel(hp2, src, dst)
    hp3 = _mid_kernel(p, hp2, dis, b2r, w3p)
    p = _agg_kernel(hp3, src, dst)
    logits = _final_kernel(p, hp3, dis, b3r, wlp, blr)
    return logits[:N]


# speedup vs baseline: 1.0259x; 1.0259x over previous
import functools

import jax
import jax.numpy as jnp
from jax import lax
from jax.experimental import pallas as pl
from jax.experimental.pallas import tpu as pltpu
from jax.experimental.pallas import tpu_sc as plsc

N = 10000
E = 320000
D_IN = 128
D_HID = 64
N_CLASSES = 4

N2 = 10240
DP = 128

NC = 2
NS = 16
NW = NC * NS
CH = 128
NCH = 80
EPT = NCH * CH
E2 = NW * EPT
RPT = N2 // NS
RCH = 40
RD = 320

_MESH = plsc.VectorSubcoreMesh(core_axis_name="c", subcore_axis_name="s")


@functools.partial(
    pl.kernel,
    out_type=jax.ShapeDtypeStruct((NC, N2, 16), jnp.float32),
    mesh=_MESH,
    scratch_types=[
        pltpu.VMEM_SHARED((N2, 16), jnp.float32),
        pltpu.VMEM((2, CH), jnp.int32),
        pltpu.VMEM((CH, 16), jnp.float32),
        pltpu.VMEM((RD, 16), jnp.float32),
        pltpu.SemaphoreType.DMA,
        pltpu.SemaphoreType.DMA,
    ],
)
def _deg_kernel(dst_hbm, out_hbm, accum, dstv, ones_rows, zbuf, sd0, sd1):
    c = lax.axis_index("c")
    s = lax.axis_index("s")
    wid = c * NS + s
    sd = (sd0, sd1)

    one = jnp.full((16,), 1.0, dtype=jnp.float32)
    zero = jnp.full((16,), 0.0, dtype=jnp.float32)

    @pl.loop(0, CH)
    def _(i):
        ones_rows[i, :] = one

    @pl.loop(0, RD)
    def _(i):
        zbuf[i, :] = zero

    base0 = wid * EPT
    for j in range(2):
        pltpu.async_copy(dst_hbm.at[pl.ds(base0 + j * CH, CH)],
                         dstv.at[j], sd[j])

    r0 = s * RPT
    for k in range(RPT // RD):
        pltpu.sync_copy(zbuf, accum.at[pl.ds(r0 + k * RD, RD)])
    plsc.subcore_barrier()

    @pl.loop(0, NCH, step=2)
    def _(g):
        for j in range(2):
            ci = g + j
            pltpu.make_async_copy(dst_hbm.at[pl.ds(0, CH)],
                                  dstv.at[j], sd[j]).wait()
            pltpu.sync_copy(ones_rows, accum.at[dstv.at[j]], add=True)

            @pl.when(ci + 2 < NCH)
            def _():
                pltpu.async_copy(
                    dst_hbm.at[pl.ds(base0 + (ci + 2) * CH, CH)],
                    dstv.at[j], sd[j])

    plsc.subcore_barrier()
    for k in range(RPT // RD):
        pltpu.sync_copy(accum.at[pl.ds(r0 + k * RD, RD)], zbuf)
        pltpu.sync_copy(zbuf, out_hbm.at[c, pl.ds(r0 + k * RD, RD)])


@functools.partial(
    pl.kernel,
    out_type=jax.ShapeDtypeStruct((NC, N2, DP), jnp.float32),
    mesh=_MESH,
    scratch_types=[
        pltpu.VMEM_SHARED((N2, DP), jnp.float32),
        pltpu.VMEM((4, CH), jnp.int32),
        pltpu.VMEM((4, CH), jnp.int32),
        pltpu.VMEM((2, CH, DP), jnp.float32),
        pltpu.VMEM((2, RCH, DP), jnp.float32),
        pltpu.SemaphoreType.DMA, pltpu.SemaphoreType.DMA,
        pltpu.SemaphoreType.DMA, pltpu.SemaphoreType.DMA,
        pltpu.SemaphoreType.DMA, pltpu.SemaphoreType.DMA,
        pltpu.SemaphoreType.DMA, pltpu.SemaphoreType.DMA,
        pltpu.SemaphoreType.DMA, pltpu.SemaphoreType.DMA,
        pltpu.SemaphoreType.DMA, pltpu.SemaphoreType.DMA,
    ],
)
def _agg_kernel(h_hbm, src_hbm, dst_hbm, out_hbm,
                accum, srcv, dstv, rows, zbuf,
                ss0, ss1, ss2, ss3, sd0, sd1, sd2, sd3,
                sg0, sg1, sw0, sw1):
    c = lax.axis_index("c")
    s = lax.axis_index("s")
    wid = c * NS + s
    ssrc = (ss0, ss1, ss2, ss3)
    sdst = (sd0, sd1, sd2, sd3)
    sgat = ((sg0,), (sg1,))
    swr = (sw0, sw1)

    zero = jnp.full((16,), 0.0, dtype=jnp.float32)

    @pl.loop(0, RCH)
    def _(i):
        @pl.loop(0, DP, step=16)
        def _(j):
            zbuf[0, i, pl.ds(j, 16)] = zero

    r0 = s * RPT
    for k in range(RPT // RCH):
        pltpu.sync_copy(zbuf.at[0], accum.at[pl.ds(r0 + k * RCH, RCH)])
    plsc.subcore_barrier()

    base0 = wid * EPT

    def idx_start(chunk, j):
        pltpu.async_copy(src_hbm.at[pl.ds(base0 + chunk * CH, CH)],
                         srcv.at[j], ssrc[j])
        pltpu.async_copy(dst_hbm.at[pl.ds(base0 + chunk * CH, CH)],
                         dstv.at[j], sdst[j])

    def src_wait(j):
        pltpu.make_async_copy(src_hbm.at[pl.ds(0, CH)],
                              srcv.at[j], ssrc[j]).wait()

    def dst_wait(j):
        pltpu.make_async_copy(dst_hbm.at[pl.ds(0, CH)],
                              dstv.at[j], sdst[j]).wait()

    def gather_start(j, r):
        pltpu.async_copy(h_hbm.at[srcv.at[j]], rows.at[r], sgat[r][0])

    def gather_wait(r):
        pltpu.make_async_copy(h_hbm.at[pl.ds(0, CH)],
                              rows.at[r], sgat[r][0]).wait()

    def scatter(j, r):
        pltpu.sync_copy(rows.at[r], accum.at[dstv.at[j]], add=True)

    for j in range(4):
        idx_start(j, j)
    src_wait(0)
    gather_start(0, 0)

    @pl.loop(0, NCH, step=4)
    def _(g):
        for j in range(4):
            ci = g + j
            jn = (j + 1) % 4
            r = j % 2
            rn = (j + 1) % 2

            @pl.when(ci + 1 < NCH)
            def _():
                src_wait(jn)
                gather_start(jn, rn)

            gather_wait(r)
            dst_wait(j)
            scatter(j, r)

            @pl.when(ci + 4 < NCH)
            def _():
                idx_start(ci + 4, j)

    plsc.subcore_barrier()
    for k in range(RPT // RCH):
        b = k % 2
        if k >= 2:
            pltpu.make_async_copy(zbuf.at[b],
                                  out_hbm.at[c, pl.ds(r0, RCH)],
                                  swr[b]).wait()
        pltpu.sync_copy(accum.at[pl.ds(r0 + k * RCH, RCH)], zbuf.at[b])
        pltpu.async_copy(zbuf.at[b],
                         out_hbm.at[c, pl.ds(r0 + k * RCH, RCH)], swr[b])
    for b in range(2):
        pltpu.make_async_copy(zbuf.at[b], out_hbm.at[c, pl.ds(r0, RCH)],
                              swr[b]).wait()


_RB = 640
_NB = N2 // _RB


def _pre_body(degp_ref, x_ref, w0_ref, dis_ref, hp_ref):
    deg = 1.0 + degp_ref[0, :, 0:1] + degp_ref[1, :, 0:1]
    dis = lax.rsqrt(deg)
    dis_ref[...] = dis
    h = jnp.dot(x_ref[...], w0_ref[...], preferred_element_type=jnp.float32)
    hp_ref[...] = h * dis


def _pre_kernel(degp, x, w0):
    return pl.pallas_call(
        _pre_body,
        grid=(_NB,),
        in_specs=[
            pl.BlockSpec((NC, _RB, 16), lambda i: (0, i, 0)),
            pl.BlockSpec((_RB, D_IN), lambda i: (i, 0)),
            pl.BlockSpec((D_IN, DP), lambda i: (0, 0)),
        ],
        out_specs=[
            pl.BlockSpec((_RB, 1), lambda i: (i, 0)),
            pl.BlockSpec((_RB, DP), lambda i: (i, 0)),
        ],
        out_shape=[
            jax.ShapeDtypeStruct((N2, 1), jnp.float32),
            jax.ShapeDtypeStruct((N2, DP), jnp.float32),
        ],
    )(degp, x, w0)


def _mid_body(p_ref, hp_ref, dis_ref, b_ref, w_ref, out_ref):
    dis = dis_ref[...]
    act = (p_ref[0] + p_ref[1] + hp_ref[...]) * dis + b_ref[...]
    act = jnp.maximum(act, 0.0)
    h = jnp.dot(act, w_ref[...], preferred_element_type=jnp.float32)
    out_ref[...] = h * dis


def _mid_kernel(p, hp, dis, b, w):
    return pl.pallas_call(
        _mid_body,
        grid=(_NB,),
        in_specs=[
            pl.BlockSpec((NC, _RB, DP), lambda i: (0, i, 0)),
            pl.BlockSpec((_RB, DP), lambda i: (i, 0)),
            pl.BlockSpec((_RB, 1), lambda i: (i, 0)),
            pl.BlockSpec((1, DP), lambda i: (0, 0)),
            pl.BlockSpec((DP, DP), lambda i: (0, 0)),
        ],
        out_specs=pl.BlockSpec((_RB, DP), lambda i: (i, 0)),
        out_shape=jax.ShapeDtypeStruct((N2, DP), jnp.float32),
    )(p, hp, dis, b, w)


def _final_body(p_ref, hp_ref, dis_ref, b_ref, wl_ref, bl_ref, out_ref):
    dis = dis_ref[...]
    act = (p_ref[0] + p_ref[1] + hp_ref[...]) * dis + b_ref[...]
    act = jnp.maximum(act, 0.0)
    logits = jnp.dot(act, wl_ref[...], preferred_element_type=jnp.float32)
    logits = logits + bl_ref[...]
    m = jnp.max(logits, axis=-1, keepdims=True)
    zs = logits - m
    lse = jnp.log(jnp.sum(jnp.exp(zs), axis=-1, keepdims=True))
    out_ref[...] = zs - lse


def _final_kernel(p, hp, dis, b, wl, bl):
    return pl.pallas_call(
        _final_body,
        grid=(_NB,),
        in_specs=[
            pl.BlockSpec((NC, _RB, DP), lambda i: (0, i, 0)),
            pl.BlockSpec((_RB, DP), lambda i: (i, 0)),
            pl.BlockSpec((_RB, 1), lambda i: (i, 0)),
            pl.BlockSpec((1, DP), lambda i: (0, 0)),
            pl.BlockSpec((DP, N_CLASSES), lambda i: (0, 0)),
            pl.BlockSpec((1, N_CLASSES), lambda i: (0, 0)),
        ],
        out_specs=pl.BlockSpec((_RB, N_CLASSES), lambda i: (i, 0)),
        out_shape=jax.ShapeDtypeStruct((N2, N_CLASSES), jnp.float32),
    )(p, hp, dis, b, wl, bl)


def _pad_w(w):
    return jnp.pad(w, ((0, DP - w.shape[0]), (0, DP - w.shape[1])))


def kernel(x, edge_index, W0, b0, W1, b1, W2, b2, W3, b3, Wl, bl):
    pad_ids = (jnp.arange(E2 - E, dtype=jnp.int32) % (N2 - N)) + N
    src = jnp.concatenate([edge_index[0], pad_ids])
    dst = jnp.concatenate([edge_index[1], pad_ids])
    xp = jnp.pad(x, ((0, N2 - N), (0, 0)))
    w0p = jnp.pad(W0, ((0, 0), (0, DP - D_HID)))
    w1p, w2p, w3p = _pad_w(W1), _pad_w(W2), _pad_w(W3)
    wlp = jnp.pad(Wl, ((0, DP - D_HID), (0, 0)))
    b0r = jnp.pad(b0, (0, DP - D_HID)).reshape(1, DP)
    b1r = jnp.pad(b1, (0, DP - D_HID)).reshape(1, DP)
    b2r = jnp.pad(b2, (0, DP - D_HID)).reshape(1, DP)
    b3r = jnp.pad(b3, (0, DP - D_HID)).reshape(1, DP)
    blr = bl.reshape(1, N_CLASSES)

    degp = _deg_kernel(dst)
    dis, hp0 = _pre_kernel(degp, xp, w0p)
    p = _agg_kernel(hp0, src, dst)
    hp1 = _mid_kernel(p, hp0, dis, b0r, w1p)
    p = _agg_kernel(hp1, src, dst)
    hp2 = _mid_kernel(p, hp1, dis, b1r, w2p)
    p = _agg_kernel(hp2, src, dst)
    hp3 = _mid_kernel(p, hp2, dis, b2r, w3p)
    p = _agg_kernel(hp3, src, dst)
    logits = _final_kernel(p, hp3, dis, b3r, wlp, blr)
    return logits[:N]
